# baseline (speedup 1.0000x reference)
import jax
import jax.numpy as jnp
from jax import lax
from jax.experimental import pallas as pl
from jax.experimental.pallas import tpu as pltpu
from jax.experimental.pallas import tpu_sc as plsc

_B = 125
_NPER = 80
_N = _B * _NPER
_E = 160000
_LCH = 256
_HALF = _LCH // 2

_W1 = 144
_NTILES = 16
_EPT = _E // _NTILES
_CH = 125
_NCH = _EPT // _CH
_NPH = 2
_CPP = _NCH // _NPH
_RING = 2

_F32 = jnp.float32


def _mmT(a, w):
    return lax.dot_general(a, w, dimension_numbers=(((1,), (1,)), ((), ())),
                           preferred_element_type=_F32)


_NB = 5
_BN = _N // _NB
_BG = _B // _NB


def _pool(n):
    return jnp.mean(jnp.reshape(n, (_BG, _NPER, n.shape[1])), axis=1)[None]


def _prep_body(x13_ref, w13_ref, b13_ref, posb_ref, w1f_ref,
               n0_ref, hd1_ref, hsa_ref, hsb_ref, pool0_ref):
    raw = _mmT(x13_ref[...], w13_ref[...]) + b13_ref[...]
    posb = posb_ref[...]
    raw = raw + jnp.reshape(jnp.broadcast_to(posb[None], (_BG, _NPER, 512)),
                            (_BN, 512))
    col = lax.broadcasted_iota(jnp.int32, raw.shape, 1)
    n0 = jnp.where(col >= 128, jnp.maximum(raw, 0.0), raw)
    n0_ref[...] = n0
    big = _mmT(n0, w1f_ref[...])
    hd1_ref[...] = big[:, :256]
    hsa_ref[...] = big[:, 256:384]
    hsb_ref[...] = big[:, 384:512]
    pool0_ref[...] = _pool(n0)


def _rows(f):
    return pl.BlockSpec((_BN, f), lambda i: (i, 0))


def _full(shape):
    return pl.BlockSpec(shape, lambda i: tuple(0 for _ in shape))


def _poolspec(f):
    return pl.BlockSpec((1, _BG, f), lambda i: (i, 0, 0))


def _prep_call(x13, w13, b13, posb, w1f):
    return pl.pallas_call(
        _prep_body,
        grid=(_NB,),
        in_specs=[_rows(13), _full((512, 13)), _full((1, 512)),
                  _full((_NPER, 512)), _full((512, 512))],
        out_specs=[_rows(512), _rows(256), _rows(_HALF), _rows(_HALF),
                   _poolspec(512)],
        out_shape=[
            jax.ShapeDtypeStruct((_N, 512), _F32),
            jax.ShapeDtypeStruct((_N, 256), _F32),
            jax.ShapeDtypeStruct((_N, _HALF), _F32),
            jax.ShapeDtypeStruct((_N, _HALF), _F32),
            jax.ShapeDtypeStruct((_NB, _BG, 512), _F32),
        ],
    )(x13, w13, b13, posb, w1f)


def _comb_body_factory(residual, has_next, wa):
    def body(*refs):
        if residual:
            hd_ref, sa_ref, sb_ref, cnt_ref, bvec_ref, prev_ref = refs[:6]
            rest = refs[6:]
        else:
            hd_ref, sa_ref, sb_ref, cnt_ref, bvec_ref = refs[:5]
            rest = refs[5:]
        if has_next:
            wn_ref = rest[0]
            n_ref, hdn_ref, hsna_ref, hsnb_ref, pool_ref = rest[1:]
        else:
            pool_ref = rest[0]
        cnt = cnt_ref[...][:, :1]
        inv = 1.0 / jnp.maximum(cnt, 1.0)
        scat = jnp.concatenate([sa_ref[...][:, :_HALF], sb_ref[...][:, :_HALF]],
                               axis=1)
        mean = hd_ref[...] + bvec_ref[...] + scat * inv
        mean = jnp.where(cnt > 0.0, mean, 0.0)
        a = jnp.maximum(mean, 0.0)
        n = prev_ref[...] + a if residual else a
        if has_next:
            n_ref[...] = n
            big = _mmT(n, wn_ref[...])
            hdn_ref[...] = big[:, :256]
            hsna_ref[...] = big[:, 256:384]
            hsnb_ref[...] = big[:, 384:512]
        pool_ref[...] = _pool(n)
    return body


def _comb_call(residual, has_next, hd, sa, sb, cnt16, bvec,
               prev=None, wn=None):
    wa = sa.shape[1]
    outs = []
    if has_next:
        outs += [jax.ShapeDtypeStruct((_N, 256), _F32),
                 jax.ShapeDtypeStruct((_N, 256), _F32),
                 jax.ShapeDtypeStruct((_N, _HALF), _F32),
                 jax.ShapeDtypeStruct((_N, _HALF), _F32)]
    outs.append(jax.ShapeDtypeStruct((_NB, _BG, 256), _F32))
    in_specs = [_rows(256), _rows(wa), _rows(wa), _rows(16),
                _full((1, 256))]
    args = [hd, sa, sb, cnt16, bvec]
    if residual:
        args.append(prev)
        in_specs.append(_rows(256))
    if has_next:
        args.append(wn)
        in_specs.append(_full((512, 256)))
    out_specs = []
    if has_next:
        out_specs += [_rows(256), _rows(256), _rows(_HALF), _rows(_HALF)]
    out_specs.append(_poolspec(256))
    return pl.pallas_call(
        _comb_body_factory(residual, has_next, wa),
        grid=(_NB,),
        in_specs=in_specs,
        out_specs=out_specs,
        out_shape=outs,
    )(*args)


def _lean_body_factory(residual):
    def body(*refs):
        if residual:
            hd_ref, sa_ref, sb_ref, cnt_ref, bvec_ref, prev_ref, ws_ref = refs[:7]
            n_ref, hsa_ref, hsb_ref = refs[7:]
        else:
            hd_ref, sa_ref, sb_ref, cnt_ref, bvec_ref, ws_ref = refs[:6]
            n_ref, hsa_ref, hsb_ref = refs[6:]
        cnt = cnt_ref[...][:, :1]
        inv = 1.0 / jnp.maximum(cnt, 1.0)
        scat = jnp.concatenate([sa_ref[...][:, :_HALF], sb_ref[...][:, :_HALF]],
                               axis=1)
        mean = hd_ref[...] + bvec_ref[...] + scat * inv
        mean = jnp.where(cnt > 0.0, mean, 0.0)
        a = jnp.maximum(mean, 0.0)
        n = prev_ref[...] + a if residual else a
        n_ref[...] = n
        bigs = _mmT(n, ws_ref[...])
        hsa_ref[...] = bigs[:, :_HALF]
        hsb_ref[...] = bigs[:, _HALF:]
    return body


def _lean_call(residual, hd, sa, sb, cnt16, bvec, ws, prev=None):
    in_specs = [_rows(256), _rows(_HALF), _rows(_HALF), _rows(16),
                _full((1, 256))]
    args = [hd, sa, sb, cnt16, bvec]
    if residual:
        args.append(prev)
        in_specs.append(_rows(256))
    args.append(ws)
    in_specs.append(_full((256, 256)))
    return pl.pallas_call(
        _lean_body_factory(residual),
        grid=(_NB,),
        in_specs=in_specs,
        out_specs=[_rows(256), _rows(_HALF), _rows(_HALF)],
        out_shape=[jax.ShapeDtypeStruct((_N, 256), _F32),
                   jax.ShapeDtypeStruct((_N, _HALF), _F32),
                   jax.ShapeDtypeStruct((_N, _HALF), _F32)],
    )(*args)


def _rest_body(n_ref, wd_ref, hdn_ref, pool_ref):
    n = n_ref[...]
    hdn_ref[...] = _mmT(n, wd_ref[...])
    pool_ref[...] = _pool(n)


def _rest_call(n, wd):
    return pl.pallas_call(
        _rest_body,
        grid=(_NB,),
        in_specs=[_rows(256), _full((256, 256))],
        out_specs=[_rows(256), _poolspec(256)],
        out_shape=[jax.ShapeDtypeStruct((_N, 256), _F32),
                   jax.ShapeDtypeStruct((_NB, _BG, 256), _F32)],
    )(n, wd)


def _head_body(p0_ref, p1_ref, p2_ref, p3_ref, aggw_ref, aggb_ref,
               muw_ref, mub_ref, varw_ref, varb_ref, mu_ref, lv_ref):
    g = jnp.concatenate([p0_ref[...], p1_ref[...], p2_ref[...], p3_ref[...]],
                        axis=1)
    zhid = _mmT(g, aggw_ref[...]) + aggb_ref[...]
    mu_ref[...] = _mmT(zhid, muw_ref[...]) + mub_ref[...]
    lv_ref[...] = _mmT(zhid, varw_ref[...]) + varb_ref[...]


def _head_call(p0, p1, p2, p3, aggw, aggb, muw, mub, varw, varb):
    return pl.pallas_call(
        _head_body,
        out_shape=[jax.ShapeDtypeStruct((_B, 256), _F32),
                   jax.ShapeDtypeStruct((_B, 256), _F32)],
    )(p0, p1, p2, p3, aggw, aggb, muw, mub, varw, varb)


def _sc_agg_call(src3, dst3, hsa, hsb, zw):
    width = hsa.shape[1]
    mesh = plsc.VectorSubcoreMesh(core_axis_name="c", subcore_axis_name="s")
    outs = [jax.ShapeDtypeStruct((_N, width), _F32),
            jax.ShapeDtypeStruct((_N, width), _F32)]
    scratch = [
        pltpu.VMEM((_CPP, _CH), jnp.int32),
        pltpu.VMEM((_CPP, _CH), jnp.int32),
        pltpu.VMEM((_RING, _CH, width), _F32),
        pltpu.VMEM_SHARED((_N, width), _F32),
    ] + [pltpu.SemaphoreType.DMA] * _RING

    def body(src3_r, dst3_r, hsa_r, hsb_r, zw_r,
             outa, outb, isrc, idst, rows, acc, *sems):
        c = lax.axis_index("c")
        s = lax.axis_index("s")
        r0 = s * 624

        @pl.when(s < 15)
        def _():
            pltpu.sync_copy(zw_r.at[pl.ds(0, 624)], acc.at[pl.ds(r0, 624)])

        @pl.when(s == 15)
        def _():
            pltpu.sync_copy(zw_r.at[pl.ds(0, 640)], acc.at[pl.ds(9360, 640)])
        plsc.subcore_barrier()

        def run(tab):
            for ph in range(_NPH):
                base = ph * _CPP
                pltpu.sync_copy(src3_r.at[s, pl.ds(base, _CPP)], isrc)
                pltpu.sync_copy(dst3_r.at[s, pl.ds(base, _CPP)], idst)
                for b in range(_RING):
                    pltpu.async_copy(tab.at[isrc.at[b]], rows.at[b], sems[b])

                def group(i, carry):
                    for b in range(_RING):
                        j = _RING * i + b
                        pltpu.make_async_copy(tab.at[isrc.at[j]], rows.at[b],
                                              sems[b]).wait()
                        pltpu.sync_copy(rows.at[b], acc.at[idst.at[j]],
                                        add=True)

                        @pl.when(j + _RING < _CPP)
                        def _():
                            pltpu.async_copy(tab.at[isrc.at[j + _RING]],
                                             rows.at[b], sems[b])
                    return carry
                lax.fori_loop(0, _CPP // _RING, group, 0)

        @pl.when(c == 0)
        def _():
            run(hsa_r)

        @pl.when(c == 1)
        def _():
            run(hsb_r)

        plsc.subcore_barrier()

        def readout(out):
            @pl.when(s < 15)
            def _():
                pltpu.sync_copy(acc.at[pl.ds(r0, 624)],
                                out.at[pl.ds(r0, 624)])

            @pl.when(s == 15)
            def _():
                pltpu.sync_copy(acc.at[pl.ds(9360, 640)],
                                out.at[pl.ds(9360, 640)])

        @pl.when(c == 0)
        def _():
            readout(outa)

        @pl.when(c == 1)
        def _():
            readout(outb)

    fn = pl.kernel(body, mesh=mesh, out_type=outs, scratch_types=scratch)
    return fn(src3, dst3, hsa, hsb, zw)


def _sc_cnt_call(dst3, z128, ones128):
    mesh = plsc.VectorSubcoreMesh(core_axis_name="c", subcore_axis_name="s")
    outs = [jax.ShapeDtypeStruct((_N, _HALF), _F32),
            jax.ShapeDtypeStruct((_N, _HALF), _F32)]
    scratch = [
        pltpu.VMEM((_NCH, _CH), jnp.int32),
        pltpu.VMEM((_CH, _HALF), _F32),
        pltpu.VMEM_SHARED((_N, _HALF), _F32),
    ]
    half = _NCH // 2

    def body(dst3_r, z128_r, ones128_r, outa, outb, idst, onesv, cacc):
        c = lax.axis_index("c")
        s = lax.axis_index("s")
        r0 = s * 624
        pltpu.sync_copy(dst3_r.at[s], idst)
        pltpu.sync_copy(ones128_r, onesv)

        @pl.when(s < 15)
        def _():
            pltpu.sync_copy(z128_r.at[pl.ds(0, 624)], cacc.at[pl.ds(r0, 624)])

        @pl.when(s == 15)
        def _():
            pltpu.sync_copy(z128_r.at[pl.ds(0, 640)],
                            cacc.at[pl.ds(9360, 640)])
        plsc.subcore_barrier()

        def cchunk(j, carry):
            pltpu.sync_copy(onesv, cacc.at[idst.at[j]], add=True)
            return carry

        @pl.when(c == 0)
        def _():
            lax.fori_loop(0, half, cchunk, 0)

        @pl.when(c == 1)
        def _():
            lax.fori_loop(half, _NCH, cchunk, 0)
        plsc.subcore_barrier()

        def readout(out):
            @pl.when(s < 15)
            def _():
                pltpu.sync_copy(cacc.at[pl.ds(r0, 624)],
                                out.at[pl.ds(r0, 624)])

            @pl.when(s == 15)
            def _():
                pltpu.sync_copy(cacc.at[pl.ds(9360, 640)],
                                out.at[pl.ds(9360, 640)])

        @pl.when(c == 0)
        def _():
            readout(outa)

        @pl.when(c == 1)
        def _():
            readout(outb)

    fn = pl.kernel(body, mesh=mesh, out_type=outs, scratch_types=scratch)
    return fn(dst3, z128, ones128)


def _sc_cnt2_call(dst_flat, z10240, ones16):
    mesh = plsc.VectorSubcoreMesh(core_axis_name="c", subcore_axis_name="s")
    outs = jax.ShapeDtypeStruct((2, _NTILES, 10240), _F32)
    scratch = [
        pltpu.VMEM((_EPT,), jnp.int32),
        pltpu.VMEM((10240,), _F32),
        pltpu.VMEM((16,), _F32),
    ]
    nv = _EPT // 16
    nv0 = (nv // 2) // 8 * 8

    def body(dst_r, z_r, ones_r, out, ibuf, cnt, onesv):
        c = lax.axis_index("c")
        s = lax.axis_index("s")
        pltpu.sync_copy(dst_r.at[pl.ds(s * _EPT, _EPT)], ibuf)
        pltpu.sync_copy(z_r, cnt)
        pltpu.sync_copy(ones_r, onesv)
        one16 = onesv[...]

        def sloop(k, carry):
            idxv = ibuf[pl.ds(pl.multiple_of(k * 16, 16), 16)]
            plsc.addupdate_scatter(cnt, [idxv], one16)
            return carry

        @pl.when(c == 0)
        def _():
            lax.fori_loop(0, nv0, sloop, 0)

        @pl.when(c == 1)
        def _():
            lax.fori_loop(nv0, nv, sloop, 0)
        pltpu.sync_copy(cnt, out.at[c, s])

    fn = pl.kernel(body, mesh=mesh, out_type=outs, scratch_types=scratch)
    return fn(dst_flat, z10240, ones16)


def kernel(x, edge_index, node_pos, node_size, b_shape, b_iou, batch,
           enc_shape_w, enc_shape_b, enc_iou_w, enc_iou_b,
           pos_w, pos_b, size_w, size_b, ex_w, ex_b, ft_w, ft_b,
           ec1_w, ec1_b, ec2_w, ec2_b, ec3_w, ec3_b,
           agg_w, agg_b, mu_w, mu_b, var_w, var_b):
    cw = ft_w[:, :64] @ ex_w
    w13 = jnp.zeros((512, 13), _F32)
    w13 = w13.at[0:64, 0:6].set(enc_shape_w)
    w13 = w13.at[64:128, 6:7].set(enc_iou_w)
    w13 = w13.at[128:256, 7:9].set(size_w)
    w13 = w13.at[256:384, 9:11].set(pos_w)
    w13 = w13.at[384:512, 11:13].set(cw)
    b13 = jnp.concatenate([enc_shape_b, enc_iou_b, size_b, pos_b, ft_b])[None]
    posb = jnp.zeros((_NPER, 512), _F32).at[:, 384:512].set(ft_w[:, 64:144].T)
    x13 = jnp.concatenate([b_shape, b_iou, node_size, node_pos, x], axis=1)

    w1f = jnp.concatenate([ec1_w[:, :512],
                           ec1_w[:128, 512:], ec1_w[128:, 512:]], axis=0)
    w2d = ec2_w[:, :256]
    w2s = jnp.concatenate([ec2_w[:128, 256:], ec2_w[128:, 256:]], axis=0)
    w3d = ec3_w[:, :256]
    w3s = jnp.concatenate([ec3_w[:128, 256:], ec3_w[128:, 256:]], axis=0)
    b1 = ec1_b[None]
    b2 = ec2_b[None]
    b3 = ec3_b[None]

    src3 = jnp.reshape(edge_index[0], (_NTILES, _NCH, _CH))
    dst3 = jnp.reshape(edge_index[1], (_NTILES, _NCH, _CH))
    z128 = jnp.zeros((640, _HALF), _F32)
    ones128 = jnp.ones((_CH, _HALF), _F32)

    cnt_a, cnt_b = _sc_cnt_call(dst3, z128, ones128)
    cnt16 = cnt_a[:, :16] + cnt_b[:, :16]
    n0, hd1, hs1a, hs1b, p0 = _prep_call(x13, w13, b13, posb, w1f)
    sc1a, sc1b = _sc_agg_call(src3, dst3, hs1a, hs1b, z128)
    n1, hs2a, hs2b = _lean_call(False, hd1, sc1a, sc1b, cnt16, b1, w2s)
    sc2a, sc2b = _sc_agg_call(src3, dst3, hs2a, hs2b, z128)
    hd2, p1 = _rest_call(n1, w2d)
    n2, hs3a, hs3b = _lean_call(True, hd2, sc2a, sc2b, cnt16, b2, w3s,
                                prev=n1)
    sc3a, sc3b = _sc_agg_call(src3, dst3, hs3a, hs3b, z128)
    hd3, p2 = _rest_call(n2, w3d)
    (p3,) = _comb_call(True, False, hd3, sc3a, sc3b, cnt16,
                       b3, prev=n2)
    p0 = jnp.reshape(p0, (_B, 512))
    p1 = jnp.reshape(p1, (_B, 256))
    p2 = jnp.reshape(p2, (_B, 256))
    p3 = jnp.reshape(p3, (_B, 256))
    mu, lv = _head_call(p0, p1, p2, p3, agg_w, agg_b[None],
                        mu_w, mu_b[None], var_w, var_b[None])
    return (mu, lv)


# speedup vs baseline: 1.0155x; 1.0155x over previous
import jax
import jax.numpy as jnp
from jax import lax
from jax.experimental import pallas as pl
from jax.experimental.pallas import tpu as pltpu
from jax.experimental.pallas import tpu_sc as plsc

_B = 125
_NPER = 80
_N = _B * _NPER
_E = 160000
_LCH = 256
_HALF = _LCH // 2

_NTILES = 16
_EPT = _E // _NTILES
_CH = 125
_NCH = _EPT // _CH
_NPH = 2
_CPP = _NCH // _NPH
_RING = 2

_F32 = jnp.float32


def _mmT(a, w):
    return lax.dot_general(a, w, dimension_numbers=(((1,), (1,)), ((), ())),
                           preferred_element_type=_F32)


_NB = 5
_BN = _N // _NB
_BG = _B // _NB


def _pool(n):
    return jnp.mean(jnp.reshape(n, (_BG, _NPER, n.shape[1])), axis=1)[None]


def _prep_body(x13_ref, w13_ref, b13_ref, posb_ref, w1f_ref,
               n0_ref, hd1_ref, hsa_ref, hsb_ref, pool0_ref):
    raw = _mmT(x13_ref[...], w13_ref[...]) + b13_ref[...]
    posb = posb_ref[...]
    raw = raw + jnp.reshape(jnp.broadcast_to(posb[None], (_BG, _NPER, 512)),
                            (_BN, 512))
    col = lax.broadcasted_iota(jnp.int32, raw.shape, 1)
    n0 = jnp.where(col >= 128, jnp.maximum(raw, 0.0), raw)
    n0_ref[...] = n0
    big = _mmT(n0, w1f_ref[...])
    hd1_ref[...] = big[:, :256]
    hsa_ref[...] = big[:, 256:384]
    hsb_ref[...] = big[:, 384:512]
    pool0_ref[...] = _pool(n0)


def _rows(f):
    return pl.BlockSpec((_BN, f), lambda i: (i, 0))


def _full(shape):
    return pl.BlockSpec(shape, lambda i: tuple(0 for _ in shape))


def _poolspec(f):
    return pl.BlockSpec((1, _BG, f), lambda i: (i, 0, 0))


def _prep_call(x13, w13, b13, posb, w1f):
    return pl.pallas_call(
        _prep_body,
        grid=(_NB,),
        in_specs=[_rows(13), _full((512, 13)), _full((1, 512)),
                  _full((_NPER, 512)), _full((512, 512))],
        out_specs=[_rows(512), _rows(256), _rows(_HALF), _rows(_HALF),
                   _poolspec(512)],
        out_shape=[
            jax.ShapeDtypeStruct((_N, 512), _F32),
            jax.ShapeDtypeStruct((_N, 256), _F32),
            jax.ShapeDtypeStruct((_N, _HALF), _F32),
            jax.ShapeDtypeStruct((_N, _HALF), _F32),
            jax.ShapeDtypeStruct((_NB, _BG, 512), _F32),
        ],
    )(x13, w13, b13, posb, w1f)


def _comb_body_factory(residual, has_next, wa):
    def body(*refs):
        if residual:
            hd_ref, sa_ref, sb_ref, cnt_ref, bvec_ref, prev_ref = refs[:6]
            rest = refs[6:]
        else:
            hd_ref, sa_ref, sb_ref, cnt_ref, bvec_ref = refs[:5]
            rest = refs[5:]
        if has_next:
            wn_ref = rest[0]
            n_ref, hdn_ref, hsna_ref, hsnb_ref, pool_ref = rest[1:]
        else:
            pool_ref = rest[0]
        cnt = cnt_ref[...][:, :1]
        inv = 1.0 / jnp.maximum(cnt, 1.0)
        scat = jnp.concatenate([sa_ref[...][:, :_HALF], sb_ref[...][:, :_HALF]],
                               axis=1)
        mean = hd_ref[...] + bvec_ref[...] + scat * inv
        mean = jnp.where(cnt > 0.0, mean, 0.0)
        a = jnp.maximum(mean, 0.0)
        n = prev_ref[...] + a if residual else a
        if has_next:
            n_ref[...] = n
            big = _mmT(n, wn_ref[...])
            hdn_ref[...] = big[:, :256]
            hsna_ref[...] = big[:, 256:384]
            hsnb_ref[...] = big[:, 384:512]
        pool_ref[...] = _pool(n)
    return body


def _comb_call(residual, has_next, hd, sa, sb, cnt16, bvec,
               prev=None, wn=None):
    wa = sa.shape[1]
    outs = []
    if has_next:
        outs += [jax.ShapeDtypeStruct((_N, 256), _F32),
                 jax.ShapeDtypeStruct((_N, 256), _F32),
                 jax.ShapeDtypeStruct((_N, _HALF), _F32),
                 jax.ShapeDtypeStruct((_N, _HALF), _F32)]
    outs.append(jax.ShapeDtypeStruct((_NB, _BG, 256), _F32))
    in_specs = [_rows(256), _rows(wa), _rows(wa), _rows(16),
                _full((1, 256))]
    args = [hd, sa, sb, cnt16, bvec]
    if residual:
        args.append(prev)
        in_specs.append(_rows(256))
    if has_next:
        args.append(wn)
        in_specs.append(_full((512, 256)))
    out_specs = []
    if has_next:
        out_specs += [_rows(256), _rows(256), _rows(_HALF), _rows(_HALF)]
    out_specs.append(_poolspec(256))
    return pl.pallas_call(
        _comb_body_factory(residual, has_next, wa),
        grid=(_NB,),
        in_specs=in_specs,
        out_specs=out_specs,
        out_shape=outs,
    )(*args)


def _lean_body_factory(residual):
    def body(*refs):
        if residual:
            hd_ref, sa_ref, sb_ref, cnt_ref, bvec_ref, prev_ref, ws_ref = refs[:7]
            n_ref, hsa_ref, hsb_ref = refs[7:]
        else:
            hd_ref, sa_ref, sb_ref, cnt_ref, bvec_ref, ws_ref = refs[:6]
            n_ref, hsa_ref, hsb_ref = refs[6:]
        cnt = cnt_ref[...][:, :1]
        inv = 1.0 / jnp.maximum(cnt, 1.0)
        scat = jnp.concatenate([sa_ref[...][:, :_HALF], sb_ref[...][:, :_HALF]],
                               axis=1)
        mean = hd_ref[...] + bvec_ref[...] + scat * inv
        mean = jnp.where(cnt > 0.0, mean, 0.0)
        a = jnp.maximum(mean, 0.0)
        n = prev_ref[...] + a if residual else a
        n_ref[...] = n
        bigs = _mmT(n, ws_ref[...])
        hsa_ref[...] = bigs[:, :_HALF]
        hsb_ref[...] = bigs[:, _HALF:]
    return body


def _lean_call(residual, hd, sa, sb, cnt16, bvec, ws, prev=None):
    in_specs = [_rows(256), _rows(_HALF), _rows(_HALF), _rows(16),
                _full((1, 256))]
    args = [hd, sa, sb, cnt16, bvec]
    if residual:
        args.append(prev)
        in_specs.append(_rows(256))
    args.append(ws)
    in_specs.append(_full((256, 256)))
    return pl.pallas_call(
        _lean_body_factory(residual),
        grid=(_NB,),
        in_specs=in_specs,
        out_specs=[_rows(256), _rows(_HALF), _rows(_HALF)],
        out_shape=[jax.ShapeDtypeStruct((_N, 256), _F32),
                   jax.ShapeDtypeStruct((_N, _HALF), _F32),
                   jax.ShapeDtypeStruct((_N, _HALF), _F32)],
    )(*args)


def _rest_body(n_ref, wd_ref, hdn_ref, pool_ref):
    n = n_ref[...]
    hdn_ref[...] = _mmT(n, wd_ref[...])
    pool_ref[...] = _pool(n)


def _rest_call(n, wd):
    return pl.pallas_call(
        _rest_body,
        grid=(_NB,),
        in_specs=[_rows(256), _full((256, 256))],
        out_specs=[_rows(256), _poolspec(256)],
        out_shape=[jax.ShapeDtypeStruct((_N, 256), _F32),
                   jax.ShapeDtypeStruct((_NB, _BG, 256), _F32)],
    )(n, wd)


def _head_body(p0_ref, p1_ref, p2_ref, p3_ref, aggw_ref, aggb_ref,
               muw_ref, mub_ref, varw_ref, varb_ref, mu_ref, lv_ref):
    g = jnp.concatenate([p0_ref[...], p1_ref[...], p2_ref[...], p3_ref[...]],
                        axis=1)
    zhid = _mmT(g, aggw_ref[...]) + aggb_ref[...]
    mu_ref[...] = _mmT(zhid, muw_ref[...]) + mub_ref[...]
    lv_ref[...] = _mmT(zhid, varw_ref[...]) + varb_ref[...]


def _head_call(p0, p1, p2, p3, aggw, aggb, muw, mub, varw, varb):
    return pl.pallas_call(
        _head_body,
        out_shape=[jax.ShapeDtypeStruct((_B, 256), _F32),
                   jax.ShapeDtypeStruct((_B, 256), _F32)],
    )(p0, p1, p2, p3, aggw, aggb, muw, mub, varw, varb)


def _sc_agg_call(src3, dst3, hsa, hsb, zw):
    width = hsa.shape[1]
    mesh = plsc.VectorSubcoreMesh(core_axis_name="c", subcore_axis_name="s")
    outs = [jax.ShapeDtypeStruct((_N, width), _F32),
            jax.ShapeDtypeStruct((_N, width), _F32)]
    scratch = [
        pltpu.VMEM((_CPP, _CH), jnp.int32),
        pltpu.VMEM((_CPP, _CH), jnp.int32),
        pltpu.VMEM((_RING, _CH, width), _F32),
        pltpu.VMEM_SHARED((_N, width), _F32),
    ] + [pltpu.SemaphoreType.DMA] * _RING

    def body(src3_r, dst3_r, hsa_r, hsb_r, zw_r,
             outa, outb, isrc, idst, rows, acc, *sems):
        c = lax.axis_index("c")
        s = lax.axis_index("s")

        @pl.when(s == 0)
        def _():
            pltpu.sync_copy(zw_r, acc)
        plsc.subcore_barrier()

        def run(tab):
            for ph in range(_NPH):
                base = ph * _CPP
                pltpu.sync_copy(src3_r.at[s, pl.ds(base, _CPP)], isrc)
                pltpu.sync_copy(dst3_r.at[s, pl.ds(base, _CPP)], idst)
                for b in range(_RING):
                    pltpu.async_copy(tab.at[isrc.at[b]], rows.at[b], sems[b])

                def group(i, carry):
                    for b in range(_RING):
                        j = _RING * i + b
                        pltpu.make_async_copy(tab.at[isrc.at[j]], rows.at[b],
                                              sems[b]).wait()
                        pltpu.sync_copy(rows.at[b], acc.at[idst.at[j]],
                                        add=True)

                        @pl.when(j + _RING < _CPP)
                        def _():
                            pltpu.async_copy(tab.at[isrc.at[j + _RING]],
                                             rows.at[b], sems[b])
                    return carry
                lax.fori_loop(0, _CPP // _RING, group, 0)

        @pl.when(c == 0)
        def _():
            run(hsa_r)

        @pl.when(c == 1)
        def _():
            run(hsb_r)

        plsc.subcore_barrier()

        @pl.when((c == 0) & (s == 0))
        def _():
            pltpu.sync_copy(acc, outa)

        @pl.when((c == 1) & (s == 0))
        def _():
            pltpu.sync_copy(acc, outb)

    fn = pl.kernel(body, mesh=mesh, out_type=outs, scratch_types=scratch)
    return fn(src3, dst3, hsa, hsb, zw)


def _sc_cnt_call(dst3, z128, ones128):
    mesh = plsc.VectorSubcoreMesh(core_axis_name="c", subcore_axis_name="s")
    outs = [jax.ShapeDtypeStruct((_N, _HALF), _F32),
            jax.ShapeDtypeStruct((_N, _HALF), _F32)]
    scratch = [
        pltpu.VMEM((_NCH, _CH), jnp.int32),
        pltpu.VMEM((_CH, _HALF), _F32),
        pltpu.VMEM_SHARED((_N, _HALF), _F32),
    ]
    half = _NCH // 2

    def body(dst3_r, z128_r, ones128_r, outa, outb, idst, onesv, cacc):
        c = lax.axis_index("c")
        s = lax.axis_index("s")
        pltpu.sync_copy(dst3_r.at[s], idst)
        pltpu.sync_copy(ones128_r, onesv)

        @pl.when(s == 0)
        def _():
            pltpu.sync_copy(z128_r, cacc)
        plsc.subcore_barrier()

        def cchunk(j, carry):
            pltpu.sync_copy(onesv, cacc.at[idst.at[j]], add=True)
            return carry

        @pl.when(c == 0)
        def _():
            lax.fori_loop(0, half, cchunk, 0)

        @pl.when(c == 1)
        def _():
            lax.fori_loop(half, _NCH, cchunk, 0)
        plsc.subcore_barrier()

        @pl.when((c == 0) & (s == 0))
        def _():
            pltpu.sync_copy(cacc, outa)

        @pl.when((c == 1) & (s == 0))
        def _():
            pltpu.sync_copy(cacc, outb)

    fn = pl.kernel(body, mesh=mesh, out_type=outs, scratch_types=scratch)
    return fn(dst3, z128, ones128)


def kernel(x, edge_index, node_pos, node_size, b_shape, b_iou, batch,
           enc_shape_w, enc_shape_b, enc_iou_w, enc_iou_b,
           pos_w, pos_b, size_w, size_b, ex_w, ex_b, ft_w, ft_b,
           ec1_w, ec1_b, ec2_w, ec2_b, ec3_w, ec3_b,
           agg_w, agg_b, mu_w, mu_b, var_w, var_b):
    cw = ft_w[:, :64] @ ex_w
    w13 = jnp.zeros((512, 13), _F32)
    w13 = w13.at[0:64, 0:6].set(enc_shape_w)
    w13 = w13.at[64:128, 6:7].set(enc_iou_w)
    w13 = w13.at[128:256, 7:9].set(size_w)
    w13 = w13.at[256:384, 9:11].set(pos_w)
    w13 = w13.at[384:512, 11:13].set(cw)
    b13 = jnp.concatenate([enc_shape_b, enc_iou_b, size_b, pos_b, ft_b])[None]
    posb = jnp.zeros((_NPER, 512), _F32).at[:, 384:512].set(ft_w[:, 64:144].T)
    x13 = jnp.concatenate([b_shape, b_iou, node_size, node_pos, x], axis=1)

    w1f = jnp.concatenate([ec1_w[:, :512],
                           ec1_w[:128, 512:], ec1_w[128:, 512:]], axis=0)
    w2d = ec2_w[:, :256]
    w2s = jnp.concatenate([ec2_w[:128, 256:], ec2_w[128:, 256:]], axis=0)
    w3d = ec3_w[:, :256]
    w3s = jnp.concatenate([ec3_w[:128, 256:], ec3_w[128:, 256:]], axis=0)
    b1 = ec1_b[None]
    b2 = ec2_b[None]
    b3 = ec3_b[None]

    src3 = jnp.reshape(edge_index[0], (_NTILES, _NCH, _CH))
    dst3 = jnp.reshape(edge_index[1], (_NTILES, _NCH, _CH))
    z128 = jnp.zeros((_N, _HALF), _F32)
    ones128 = jnp.ones((_CH, _HALF), _F32)

    cnt_a, cnt_b = _sc_cnt_call(dst3, z128, ones128)
    cnt16 = cnt_a[:, :16] + cnt_b[:, :16]
    n0, hd1, hs1a, hs1b, p0 = _prep_call(x13, w13, b13, posb, w1f)
    sc1a, sc1b = _sc_agg_call(src3, dst3, hs1a, hs1b, z128)
    n1, hs2a, hs2b = _lean_call(False, hd1, sc1a, sc1b, cnt16, b1, w2s)
    sc2a, sc2b = _sc_agg_call(src3, dst3, hs2a, hs2b, z128)
    hd2, p1 = _rest_call(n1, w2d)
    n2, hs3a, hs3b = _lean_call(True, hd2, sc2a, sc2b, cnt16, b2, w3s,
                                prev=n1)
    sc3a, sc3b = _sc_agg_call(src3, dst3, hs3a, hs3b, z128)
    hd3, p2 = _rest_call(n2, w3d)
    (p3,) = _comb_call(True, False, hd3, sc3a, sc3b, cnt16,
                       b3, prev=n2)
    p0 = jnp.reshape(p0, (_B, 512))
    p1 = jnp.reshape(p1, (_B, 256))
    p2 = jnp.reshape(p2, (_B, 256))
    p3 = jnp.reshape(p3, (_B, 256))
    mu, lv = _head_call(p0, p1, p2, p3, agg_w, agg_b[None],
                        mu_w, mu_b[None], var_w, var_b[None])
    return (mu, lv)


# speedup vs baseline: 1.0213x; 1.0057x over previous
import jax
import jax.numpy as jnp
from jax import lax
from jax.experimental import pallas as pl
from jax.experimental.pallas import tpu as pltpu
from jax.experimental.pallas import tpu_sc as plsc

_B = 125
_NPER = 80
_N = _B * _NPER
_E = 160000
_LCH = 256
_HALF = _LCH // 2

_NTILES = 16
_EPT = _E // _NTILES
_CH = 125
_NCH = _EPT // _CH
_NPH = 2
_CPP = _NCH // _NPH
_RING = 2

_F32 = jnp.float32


def _mmT(a, w):
    return lax.dot_general(a, w, dimension_numbers=(((1,), (1,)), ((), ())),
                           preferred_element_type=_F32)


_NB = 5
_BN = _N // _NB
_BG = _B // _NB


def _pool(n):
    return jnp.mean(jnp.reshape(n, (_BG, _NPER, n.shape[1])), axis=1)[None]


def _prep_body(x13_ref, w13_ref, b13_ref, posb_ref, w1f_ref,
               n0_ref, hd1_ref, hsa_ref, hsb_ref, pool0_ref):
    raw = _mmT(x13_ref[...], w13_ref[...]) + b13_ref[...]
    posb = posb_ref[...]
    raw = raw + jnp.reshape(jnp.broadcast_to(posb[None], (_BG, _NPER, 512)),
                            (_BN, 512))
    col = lax.broadcasted_iota(jnp.int32, raw.shape, 1)
    n0 = jnp.where(col >= 128, jnp.maximum(raw, 0.0), raw)
    n0_ref[...] = n0
    big = _mmT(n0, w1f_ref[...])
    hd1_ref[...] = big[:, :256]
    hsa_ref[...] = big[:, 256:384]
    hsb_ref[...] = big[:, 384:512]
    pool0_ref[...] = _pool(n0)


def _rows(f):
    return pl.BlockSpec((_BN, f), lambda i: (i, 0))


def _full(shape):
    return pl.BlockSpec(shape, lambda i: tuple(0 for _ in shape))


def _poolspec(f):
    return pl.BlockSpec((1, _BG, f), lambda i: (i, 0, 0))


def _prep_call(x13, w13, b13, posb, w1f):
    return pl.pallas_call(
        _prep_body,
        grid=(_NB,),
        in_specs=[_rows(13), _full((512, 13)), _full((1, 512)),
                  _full((_NPER, 512)), _full((512, 512))],
        out_specs=[_rows(512), _rows(256), _rows(_HALF), _rows(_HALF),
                   _poolspec(512)],
        out_shape=[
            jax.ShapeDtypeStruct((_N, 512), _F32),
            jax.ShapeDtypeStruct((_N, 256), _F32),
            jax.ShapeDtypeStruct((_N, _HALF), _F32),
            jax.ShapeDtypeStruct((_N, _HALF), _F32),
            jax.ShapeDtypeStruct((_NB, _BG, 512), _F32),
        ],
    )(x13, w13, b13, posb, w1f)


def _comb_body_factory(residual, has_next, wa):
    def body(*refs):
        if residual:
            hd_ref, sa_ref, sb_ref, cnt_ref, bvec_ref, prev_ref = refs[:6]
            rest = refs[6:]
        else:
            hd_ref, sa_ref, sb_ref, cnt_ref, bvec_ref = refs[:5]
            rest = refs[5:]
        if has_next:
            wn_ref = rest[0]
            n_ref, hdn_ref, hsna_ref, hsnb_ref, pool_ref = rest[1:]
        else:
            pool_ref = rest[0]
        cnt = cnt_ref[...][:, :1]
        inv = 1.0 / jnp.maximum(cnt, 1.0)
        scat = jnp.concatenate([sa_ref[...][:, :_HALF], sb_ref[...][:, :_HALF]],
                               axis=1)
        mean = hd_ref[...] + bvec_ref[...] + scat * inv
        mean = jnp.where(cnt > 0.0, mean, 0.0)
        a = jnp.maximum(mean, 0.0)
        n = prev_ref[...] + a if residual else a
        if has_next:
            n_ref[...] = n
            big = _mmT(n, wn_ref[...])
            hdn_ref[...] = big[:, :256]
            hsna_ref[...] = big[:, 256:384]
            hsnb_ref[...] = big[:, 384:512]
        pool_ref[...] = _pool(n)
    return body


def _comb_call(residual, has_next, hd, sa, sb, cnt16, bvec,
               prev=None, wn=None):
    wa = sa.shape[1]
    outs = []
    if has_next:
        outs += [jax.ShapeDtypeStruct((_N, 256), _F32),
                 jax.ShapeDtypeStruct((_N, 256), _F32),
                 jax.ShapeDtypeStruct((_N, _HALF), _F32),
                 jax.ShapeDtypeStruct((_N, _HALF), _F32)]
    outs.append(jax.ShapeDtypeStruct((_NB, _BG, 256), _F32))
    in_specs = [_rows(256), _rows(wa), _rows(wa), _rows(16),
                _full((1, 256))]
    args = [hd, sa, sb, cnt16, bvec]
    if residual:
        args.append(prev)
        in_specs.append(_rows(256))
    if has_next:
        args.append(wn)
        in_specs.append(_full((512, 256)))
    out_specs = []
    if has_next:
        out_specs += [_rows(256), _rows(256), _rows(_HALF), _rows(_HALF)]
    out_specs.append(_poolspec(256))
    return pl.pallas_call(
        _comb_body_factory(residual, has_next, wa),
        grid=(_NB,),
        in_specs=in_specs,
        out_specs=out_specs,
        out_shape=outs,
    )(*args)


def _lean_body_factory(residual):
    def body(*refs):
        if residual:
            hd_ref, sa_ref, sb_ref, cnt_ref, bvec_ref, prev_ref, ws_ref = refs[:7]
            n_ref, hsa_ref, hsb_ref = refs[7:]
        else:
            hd_ref, sa_ref, sb_ref, cnt_ref, bvec_ref, ws_ref = refs[:6]
            n_ref, hsa_ref, hsb_ref = refs[6:]
        cnt = cnt_ref[...][:, :1]
        inv = 1.0 / jnp.maximum(cnt, 1.0)
        scat = jnp.concatenate([sa_ref[...][:, :_HALF], sb_ref[...][:, :_HALF]],
                               axis=1)
        mean = hd_ref[...] + bvec_ref[...] + scat * inv
        mean = jnp.where(cnt > 0.0, mean, 0.0)
        a = jnp.maximum(mean, 0.0)
        n = prev_ref[...] + a if residual else a
        n_ref[...] = n
        bigs = _mmT(n, ws_ref[...])
        hsa_ref[...] = bigs[:, :_HALF]
        hsb_ref[...] = bigs[:, _HALF:]
    return body


def _lean_call(residual, hd, sa, sb, cnt16, bvec, ws, prev=None):
    in_specs = [_rows(256), _rows(_HALF), _rows(_HALF), _rows(16),
                _full((1, 256))]
    args = [hd, sa, sb, cnt16, bvec]
    if residual:
        args.append(prev)
        in_specs.append(_rows(256))
    args.append(ws)
    in_specs.append(_full((256, 256)))
    return pl.pallas_call(
        _lean_body_factory(residual),
        grid=(_NB,),
        in_specs=in_specs,
        out_specs=[_rows(256), _rows(_HALF), _rows(_HALF)],
        out_shape=[jax.ShapeDtypeStruct((_N, 256), _F32),
                   jax.ShapeDtypeStruct((_N, _HALF), _F32),
                   jax.ShapeDtypeStruct((_N, _HALF), _F32)],
    )(*args)


def _rest_body(n_ref, wd_ref, hdn_ref, pool_ref):
    n = n_ref[...]
    hdn_ref[...] = _mmT(n, wd_ref[...])
    pool_ref[...] = _pool(n)


def _rest_call(n, wd):
    return pl.pallas_call(
        _rest_body,
        grid=(_NB,),
        in_specs=[_rows(256), _full((256, 256))],
        out_specs=[_rows(256), _poolspec(256)],
        out_shape=[jax.ShapeDtypeStruct((_N, 256), _F32),
                   jax.ShapeDtypeStruct((_NB, _BG, 256), _F32)],
    )(n, wd)


def _tail_body(hd_ref, sa_ref, sb_ref, cnt_ref, bvec_ref, prev_ref,
               p0_ref, p1_ref, p2_ref, aggw_ref, aggb_ref,
               muw_ref, mub_ref, varw_ref, varb_ref,
               mu_ref, lv_ref, p3_acc):
    i = pl.program_id(0)

    @pl.when(i < _NB)
    def _():
        cnt = cnt_ref[...][:, :1]
        inv = 1.0 / jnp.maximum(cnt, 1.0)
        scat = jnp.concatenate([sa_ref[...], sb_ref[...]], axis=1)
        mean = hd_ref[...] + bvec_ref[...] + scat * inv
        mean = jnp.where(cnt > 0.0, mean, 0.0)
        n = prev_ref[...] + jnp.maximum(mean, 0.0)
        p3_acc[i] = _pool(n)[0]

    @pl.when(i == _NB)
    def _():
        p3 = jnp.reshape(p3_acc[...], (_B, 256))
        g = jnp.concatenate([p0_ref[...], p1_ref[...], p2_ref[...],
                             p3], axis=1)
        zhid = _mmT(g, aggw_ref[...]) + aggb_ref[...]
        mu_ref[...] = _mmT(zhid, muw_ref[...]) + mub_ref[...]
        lv_ref[...] = _mmT(zhid, varw_ref[...]) + varb_ref[...]


def _tail_call(hd, sa, sb, cnt16, bvec, prev, p0, p1, p2,
               aggw, aggb, muw, mub, varw, varb):
    def blk(f):
        return pl.BlockSpec((_BN, f), lambda i: (jnp.minimum(i, _NB - 1), 0))

    def whole(shape):
        return pl.BlockSpec(shape, lambda i: tuple(0 for _ in shape))

    return pl.pallas_call(
        _tail_body,
        grid=(_NB + 1,),
        in_specs=[blk(256), blk(_HALF), blk(_HALF), blk(16), whole((1, 256)),
                  blk(256), whole((_B, 512)), whole((_B, 256)),
                  whole((_B, 256)), whole((256, 1280)), whole((1, 256)),
                  whole((256, 256)), whole((1, 256)), whole((256, 256)),
                  whole((1, 256))],
        out_specs=[whole((_B, 256)), whole((_B, 256))],
        out_shape=[jax.ShapeDtypeStruct((_B, 256), _F32),
                   jax.ShapeDtypeStruct((_B, 256), _F32)],
        scratch_shapes=[pltpu.VMEM((_NB, _BG, 256), _F32)],
    )(hd, sa, sb, cnt16, bvec, prev, p0, p1, p2,
      aggw, aggb, muw, mub, varw, varb)


def _head_body(p0_ref, p1_ref, p2_ref, p3_ref, aggw_ref, aggb_ref,
               muw_ref, mub_ref, varw_ref, varb_ref, mu_ref, lv_ref):
    g = jnp.concatenate([p0_ref[...], p1_ref[...], p2_ref[...], p3_ref[...]],
                        axis=1)
    zhid = _mmT(g, aggw_ref[...]) + aggb_ref[...]
    mu_ref[...] = _mmT(zhid, muw_ref[...]) + mub_ref[...]
    lv_ref[...] = _mmT(zhid, varw_ref[...]) + varb_ref[...]


def _head_call(p0, p1, p2, p3, aggw, aggb, muw, mub, varw, varb):
    return pl.pallas_call(
        _head_body,
        out_shape=[jax.ShapeDtypeStruct((_B, 256), _F32),
                   jax.ShapeDtypeStruct((_B, 256), _F32)],
    )(p0, p1, p2, p3, aggw, aggb, muw, mub, varw, varb)


def _sc_agg_call(src3, dst3, hsa, hsb, zw):
    width = hsa.shape[1]
    mesh = plsc.VectorSubcoreMesh(core_axis_name="c", subcore_axis_name="s")
    outs = [jax.ShapeDtypeStruct((_N, width), _F32),
            jax.ShapeDtypeStruct((_N, width), _F32)]
    scratch = [
        pltpu.VMEM((_CPP, _CH), jnp.int32),
        pltpu.VMEM((_CPP, _CH), jnp.int32),
        pltpu.VMEM((_RING, _CH, width), _F32),
        pltpu.VMEM_SHARED((_N, width), _F32),
    ] + [pltpu.SemaphoreType.DMA] * _RING

    def body(src3_r, dst3_r, hsa_r, hsb_r, zw_r,
             outa, outb, isrc, idst, rows, acc, *sems):
        c = lax.axis_index("c")
        s = lax.axis_index("s")

        @pl.when(s == 0)
        def _():
            pltpu.sync_copy(zw_r, acc)
        plsc.subcore_barrier()

        def run(tab):
            for ph in range(_NPH):
                base = ph * _CPP
                pltpu.sync_copy(src3_r.at[s, pl.ds(base, _CPP)], isrc)
                pltpu.sync_copy(dst3_r.at[s, pl.ds(base, _CPP)], idst)
                for b in range(_RING):
                    pltpu.async_copy(tab.at[isrc.at[b]], rows.at[b], sems[b])

                def group(i, carry):
                    for b in range(_RING):
                        j = _RING * i + b
                        pltpu.make_async_copy(tab.at[isrc.at[j]], rows.at[b],
                                              sems[b]).wait()
                        pltpu.sync_copy(rows.at[b], acc.at[idst.at[j]],
                                        add=True)

                        @pl.when(j + _RING < _CPP)
                        def _():
                            pltpu.async_copy(tab.at[isrc.at[j + _RING]],
                                             rows.at[b], sems[b])
                    return carry
                lax.fori_loop(0, _CPP // _RING, group, 0)

        @pl.when(c == 0)
        def _():
            run(hsa_r)

        @pl.when(c == 1)
        def _():
            run(hsb_r)

        plsc.subcore_barrier()

        @pl.when((c == 0) & (s == 0))
        def _():
            pltpu.sync_copy(acc, outa)

        @pl.when((c == 1) & (s == 0))
        def _():
            pltpu.sync_copy(acc, outb)

    fn = pl.kernel(body, mesh=mesh, out_type=outs, scratch_types=scratch)
    return fn(src3, dst3, hsa, hsb, zw)


def _sc_cnt_call(dst3, z128, ones128):
    mesh = plsc.VectorSubcoreMesh(core_axis_name="c", subcore_axis_name="s")
    outs = [jax.ShapeDtypeStruct((_N, _HALF), _F32),
            jax.ShapeDtypeStruct((_N, _HALF), _F32)]
    scratch = [
        pltpu.VMEM((_NCH, _CH), jnp.int32),
        pltpu.VMEM((_CH, _HALF), _F32),
        pltpu.VMEM_SHARED((_N, _HALF), _F32),
    ]
    half = _NCH // 2

    def body(dst3_r, z128_r, ones128_r, outa, outb, idst, onesv, cacc):
        c = lax.axis_index("c")
        s = lax.axis_index("s")
        pltpu.sync_copy(dst3_r.at[s], idst)
        pltpu.sync_copy(ones128_r, onesv)

        @pl.when(s == 0)
        def _():
            pltpu.sync_copy(z128_r, cacc)
        plsc.subcore_barrier()

        def cchunk(j, carry):
            pltpu.sync_copy(onesv, cacc.at[idst.at[j]], add=True)
            return carry

        @pl.when(c == 0)
        def _():
            lax.fori_loop(0, half, cchunk, 0)

        @pl.when(c == 1)
        def _():
            lax.fori_loop(half, _NCH, cchunk, 0)
        plsc.subcore_barrier()

        @pl.when((c == 0) & (s == 0))
        def _():
            pltpu.sync_copy(cacc, outa)

        @pl.when((c == 1) & (s == 0))
        def _():
            pltpu.sync_copy(cacc, outb)

    fn = pl.kernel(body, mesh=mesh, out_type=outs, scratch_types=scratch)
    return fn(dst3, z128, ones128)


def kernel(x, edge_index, node_pos, node_size, b_shape, b_iou, batch,
           enc_shape_w, enc_shape_b, enc_iou_w, enc_iou_b,
           pos_w, pos_b, size_w, size_b, ex_w, ex_b, ft_w, ft_b,
           ec1_w, ec1_b, ec2_w, ec2_b, ec3_w, ec3_b,
           agg_w, agg_b, mu_w, mu_b, var_w, var_b):
    cw = ft_w[:, :64] @ ex_w
    w13 = jnp.zeros((512, 13), _F32)
    w13 = w13.at[0:64, 0:6].set(enc_shape_w)
    w13 = w13.at[64:128, 6:7].set(enc_iou_w)
    w13 = w13.at[128:256, 7:9].set(size_w)
    w13 = w13.at[256:384, 9:11].set(pos_w)
    w13 = w13.at[384:512, 11:13].set(cw)
    b13 = jnp.concatenate([enc_shape_b, enc_iou_b, size_b, pos_b, ft_b])[None]
    posb = jnp.zeros((_NPER, 512), _F32).at[:, 384:512].set(ft_w[:, 64:144].T)
    x13 = jnp.concatenate([b_shape, b_iou, node_size, node_pos, x], axis=1)

    w1f = jnp.concatenate([ec1_w[:, :512],
                           ec1_w[:128, 512:], ec1_w[128:, 512:]], axis=0)
    w2d = ec2_w[:, :256]
    w2s = jnp.concatenate([ec2_w[:128, 256:], ec2_w[128:, 256:]], axis=0)
    w3d = ec3_w[:, :256]
    w3s = jnp.concatenate([ec3_w[:128, 256:], ec3_w[128:, 256:]], axis=0)
    b1 = ec1_b[None]
    b2 = ec2_b[None]
    b3 = ec3_b[None]

    src3 = jnp.reshape(edge_index[0], (_NTILES, _NCH, _CH))
    dst3 = jnp.reshape(edge_index[1], (_NTILES, _NCH, _CH))
    z128 = jnp.zeros((_N, _HALF), _F32)
    ones128 = jnp.ones((_CH, _HALF), _F32)

    cnt_a, cnt_b = _sc_cnt_call(dst3, z128, ones128)
    cnt16 = cnt_a[:, :16] + cnt_b[:, :16]
    n0, hd1, hs1a, hs1b, p0 = _prep_call(x13, w13, b13, posb, w1f)
    sc1a, sc1b = _sc_agg_call(src3, dst3, hs1a, hs1b, z128)
    n1, hs2a, hs2b = _lean_call(False, hd1, sc1a, sc1b, cnt16, b1, w2s)
    sc2a, sc2b = _sc_agg_call(src3, dst3, hs2a, hs2b, z128)
    hd2, p1 = _rest_call(n1, w2d)
    n2, hs3a, hs3b = _lean_call(True, hd2, sc2a, sc2b, cnt16, b2, w3s,
                                prev=n1)
    sc3a, sc3b = _sc_agg_call(src3, dst3, hs3a, hs3b, z128)
    hd3, p2 = _rest_call(n2, w3d)
    p0 = jnp.reshape(p0, (_B, 512))
    p1 = jnp.reshape(p1, (_B, 256))
    p2 = jnp.reshape(p2, (_B, 256))
    mu, lv = _tail_call(hd3, sc3a, sc3b, cnt16, b3, n2, p0, p1, p2,
                        agg_w, agg_b[None], mu_w, mu_b[None],
                        var_w, var_b[None])
    return (mu, lv)
